# baseline (speedup 1.0000x reference)
import functools

import jax
import jax.numpy as jnp
from jax import lax
from jax.experimental import pallas as pl
from jax.experimental.pallas import tpu as pltpu
from jax.experimental.pallas import tpu_sc as plsc

BATCH = 16384
D = 16

_info = plsc.get_sparse_core_info()
NC, NS = _info.num_cores, _info.num_subcores
NW = NC * NS
BPW = BATCH // NW

_mesh = plsc.VectorSubcoreMesh(core_axis_name="c", subcore_axis_name="s")


@functools.partial(
    pl.kernel,
    mesh=_mesh,
    out_type=jax.ShapeDtypeStruct((BATCH, 3 * D), jnp.float32),
    compiler_params=pltpu.CompilerParams(use_tc_tiling_on_sc=False),
    scratch_types=[
        pltpu.VMEM((BPW,), jnp.int32),
        pltpu.VMEM((BPW, D), jnp.float32),
        pltpu.SemaphoreType.DMA,
    ],
)
def _gather3(uid, iid, tid, ut, it, tt, out, idx_v, rows_v, sem):
    wid = lax.axis_index("s") * NC + lax.axis_index("c")
    base = wid * BPW
    for c, (ids, table) in enumerate(((uid, ut), (iid, it), (tid, tt))):
        pltpu.sync_copy(ids.at[pl.ds(base, BPW)], idx_v)
        pltpu.async_copy(table.at[idx_v], rows_v, sem).wait()
        pltpu.sync_copy(rows_v, out.at[pl.ds(base, BPW), pl.ds(c * D, D)])


def kernel(user_id, item_id, tag_id, user_table, item_table, tag_table):
    return _gather3(user_id, item_id, tag_id, user_table, item_table, tag_table)


# speedup vs baseline: 3.5199x; 3.5199x over previous
import functools

import jax
import jax.numpy as jnp
from jax import lax
from jax.experimental import pallas as pl
from jax.experimental.pallas import tpu as pltpu
from jax.experimental.pallas import tpu_sc as plsc

BATCH = 16384
D = 16
OUT_W = 3 * D

_info = plsc.get_sparse_core_info()
NC, NS = _info.num_cores, _info.num_subcores
NW = NC * NS
BPW = BATCH // NW

NBUF = 16
NGRP = BPW // NBUF

_mesh = plsc.VectorSubcoreMesh(core_axis_name="c", subcore_axis_name="s")


@functools.partial(
    pl.kernel,
    mesh=_mesh,
    out_type=jax.ShapeDtypeStruct((BATCH * OUT_W,), jnp.float32),
    compiler_params=pltpu.CompilerParams(needs_layout_passes=False),
    scratch_types=[
        pltpu.VMEM((BPW,), jnp.int32),
        pltpu.VMEM((BPW,), jnp.int32),
        pltpu.VMEM((BPW,), jnp.int32),
        pltpu.VMEM((NBUF, D, 128), jnp.float32),
        pltpu.VMEM((BPW * OUT_W,), jnp.float32),
        [pltpu.SemaphoreType.DMA] * NBUF,
    ],
)
def _gather3(uid, iid, tid, ut, it, tt, out, uix, iix, tix, tile_v, cat_v, sems):
    wid = lax.axis_index("s") * NC + lax.axis_index("c")
    base = wid * BPW
    for ids, ivec in ((uid, uix), (iid, iix), (tid, tix)):
        pltpu.sync_copy(ids.at[pl.ds(base, BPW)], ivec)
    d_iota = lax.iota(jnp.int32, 16)

    def pick(vec, s):
        return jnp.sum(jnp.where(d_iota == s, vec, 0))

    for t, (tab, ivec) in enumerate(((ut, uix), (it, iix), (tt, tix))):

        def fire_group(g, tab=tab, ivec=ivec):
            vg = ivec[pl.ds(g * NBUF, NBUF)]
            cols = (vg >> 7) * 128
            for s in range(NBUF):
                col = pl.multiple_of(pick(cols, s), 128)
                pltpu.async_copy(
                    tab.at[:, pl.ds(col, 128)], tile_v.at[s], sems[s]
                )

        fire_group(0)

        def ring_body(g, tab=tab, t=t, ivec=ivec):
            vg = ivec[pl.ds(g * NBUF, NBUF)]
            lanes = vg & 127

            def drain_extract(s):
                pltpu.make_async_copy(
                    tab.at[:, pl.ds(0, 128)], tile_v.at[s], sems[s]
                ).wait()
                lane = jnp.broadcast_to(pick(lanes, s), (16,))
                row = plsc.load_gather(tile_v.at[s], [d_iota, lane])
                cat_v[pl.ds((g * NBUF + s) * OUT_W + t * D, D)] = row

            @pl.when(g + 1 < NGRP)
            def _():
                vn = ivec[pl.ds((g + 1) * NBUF, NBUF)]
                cols = (vn >> 7) * 128
                for s in range(NBUF):
                    drain_extract(s)
                    col = pl.multiple_of(pick(cols, s), 128)
                    pltpu.async_copy(
                        tab.at[:, pl.ds(col, 128)], tile_v.at[s], sems[s]
                    )

            @pl.when(g + 1 >= NGRP)
            def _():
                for s in range(NBUF):
                    drain_extract(s)

        pl.loop(0, NGRP)(ring_body)

    pltpu.sync_copy(cat_v, out.at[pl.ds(base * OUT_W, BPW * OUT_W)])


def kernel(user_id, item_id, tag_id, user_table, item_table, tag_table):
    flat = _gather3(
        user_id, item_id, tag_id,
        user_table.T, item_table.T, tag_table.T,
    )
    return flat.reshape(BATCH, OUT_W)


# speedup vs baseline: 3.5468x; 1.0077x over previous
import functools

import jax
import jax.numpy as jnp
from jax import lax
from jax.experimental import pallas as pl
from jax.experimental.pallas import tpu as pltpu
from jax.experimental.pallas import tpu_sc as plsc

BATCH = 16384
D = 16
OUT_W = 3 * D

_info = plsc.get_sparse_core_info()
NC, NS = _info.num_cores, _info.num_subcores
NW = NC * NS
BPW = BATCH // NW

NBUF = 16
NGRP = BPW // NBUF

_mesh = plsc.VectorSubcoreMesh(core_axis_name="c", subcore_axis_name="s")


@functools.partial(
    pl.kernel,
    mesh=_mesh,
    out_type=jax.ShapeDtypeStruct((BATCH * OUT_W,), jnp.float32),
    compiler_params=pltpu.CompilerParams(needs_layout_passes=False),
    scratch_types=[
        pltpu.VMEM((BPW,), jnp.int32),
        pltpu.VMEM((BPW,), jnp.int32),
        pltpu.VMEM((BPW,), jnp.int32),
        pltpu.VMEM((NBUF, D, 128), jnp.float32),
        pltpu.VMEM((BPW * OUT_W,), jnp.float32),
        [pltpu.SemaphoreType.DMA] * NBUF,
    ],
)
def _gather3(uid, iid, tid, ut, it, tt, out, uix, iix, tix, tile_v, cat_v, sems):
    wid = lax.axis_index("s") * NC + lax.axis_index("c")
    base = wid * BPW
    for ids, ivec in ((uid, uix), (iid, iix), (tid, tix)):
        pltpu.sync_copy(ids.at[pl.ds(base, BPW)], ivec)
    d_iota = lax.iota(jnp.int32, 16)

    def pick(vec, s):
        return vec[s]

    for t, (tab, ivec) in enumerate(((ut, uix), (it, iix), (tt, tix))):

        def fire_group(g, tab=tab, ivec=ivec):
            vg = ivec[pl.ds(g * NBUF, NBUF)]
            cols = (vg >> 7) * 128
            for s in range(NBUF):
                col = pl.multiple_of(pick(cols, s), 128)
                pltpu.async_copy(
                    tab.at[:, pl.ds(col, 128)], tile_v.at[s], sems[s]
                )

        fire_group(0)

        def ring_body(g, tab=tab, t=t, ivec=ivec):
            vg = ivec[pl.ds(g * NBUF, NBUF)]
            lanes = vg & 127

            def extract(s):
                lane = jnp.broadcast_to(pick(lanes, s), (16,))
                row = plsc.load_gather(tile_v.at[s], [d_iota, lane])
                cat_v[pl.ds((g * NBUF + s) * OUT_W + t * D, D)] = row

            def wait_slot(s):
                pltpu.make_async_copy(
                    tab.at[:, pl.ds(0, 128)], tile_v.at[s], sems[s]
                ).wait()

            @pl.when(g + 1 < NGRP)
            def _():
                vn = ivec[pl.ds((g + 1) * NBUF, NBUF)]
                cols = (vn >> 7) * 128
                for s in range(NBUF):
                    wait_slot(s)
                    extract(s)
                    col = pl.multiple_of(pick(cols, s), 128)
                    pltpu.async_copy(
                        tab.at[:, pl.ds(col, 128)], tile_v.at[s], sems[s]
                    )

            @pl.when(g + 1 >= NGRP)
            def _():
                for s in range(NBUF):
                    wait_slot(s)
                    extract(s)

        pl.loop(0, NGRP)(ring_body)

    pltpu.sync_copy(cat_v, out.at[pl.ds(base * OUT_W, BPW * OUT_W)])


def kernel(user_id, item_id, tag_id, user_table, item_table, tag_table):
    flat = _gather3(
        user_id, item_id, tag_id,
        user_table.T, item_table.T, tag_table.T,
    )
    return flat.reshape(BATCH, OUT_W)
